# Initial kernel scaffold; baseline (speedup 1.0000x reference)
import jax
import jax.numpy as jnp
from jax.experimental import pallas as pl


def kernel(hm, wh, reg, gt_hm, gt_wh, gt_reg, reg_mask, target_box, ind):
    raise NotImplementedError("write your pallas kernel here")


# speedup vs baseline: 1.0179x; 1.0179x over previous
import functools

import jax
import jax.numpy as jnp
from jax.experimental import pallas as pl
from jax.experimental.pallas import tpu as pltpu

B, C, H, W = 16, 80, 128, 128
K = 100
MAX_OBJS = 128
M = 512

PLANES = B * C
BLK = 16


def _sigmoid(x):
    return jnp.clip(jax.nn.sigmoid(x), 1e-4, 1.0 - 1e-4)


def _focal_kernel(pred_ref, gt_ref, out_ref):
    i = pl.program_id(0)

    p = pred_ref[...]
    g = gt_ref[...]
    pos = (g == 1.0).astype(jnp.float32)
    neg = 1.0 - pos
    one_m_g = 1.0 - g
    nw = one_m_g * one_m_g
    nw = nw * nw
    pos_loss = jnp.log(p) * (1.0 - p) * (1.0 - p) * pos
    neg_loss = jnp.log(1.0 - p) * p * p * nw * neg
    def r(x):
        s = jnp.sum(x, axis=0)
        return jnp.sum(s.reshape(16, 8, 128), axis=0)

    part = jnp.stack([r(pos_loss), r(neg_loss), r(pos)], axis=0)

    @pl.when(i == 0)
    def _():
        out_ref[...] = jnp.zeros_like(out_ref)

    out_ref[...] += part


def _focal_sums(pred, gt):
    out = pl.pallas_call(
        _focal_kernel,
        grid=(PLANES // BLK,),
        in_specs=[
            pl.BlockSpec((BLK, H, W), lambda i: (i, 0, 0)),
            pl.BlockSpec((BLK, H, W), lambda i: (i, 0, 0)),
        ],
        out_specs=pl.BlockSpec((3, 8, 128), lambda i: (0, 0, 0)),
        out_shape=jax.ShapeDtypeStruct((3, 8, 128), jnp.float32),
    )(pred.reshape(PLANES, H, W), gt.reshape(PLANES, H, W))
    sums = jnp.sum(out, axis=(1, 2))
    return sums[0], sums[1], sums[2]


def _nms(heat):
    hmax = jax.lax.reduce_window(heat, -jnp.inf, jax.lax.max, (1, 1, 3, 3), (1, 1, 1, 1), "SAME")
    keep = (hmax == heat).astype(heat.dtype)
    return heat * keep


def _gather_feat(feat, ind):
    b, n, c = feat.shape[0], ind.shape[1], feat.shape[2]
    idx = jnp.broadcast_to(ind[:, :, None], (b, n, c))
    return jnp.take_along_axis(feat, idx, axis=1)


def _transpose_and_gather_feat(feat, ind):
    b, c, h, w = feat.shape
    f = jnp.transpose(feat, (0, 2, 3, 1)).reshape(b, h * w, c)
    return _gather_feat(f, ind)


def _topk(scores, k):
    b, c, h, w = scores.shape
    topk_scores, topk_inds = jax.lax.top_k(scores.reshape(b, c, -1), k)
    topk_inds = topk_inds % (h * w)
    topk_ys = (topk_inds // w).astype(jnp.float32)
    topk_xs = (topk_inds % w).astype(jnp.float32)
    topk_score, topk_ind = jax.lax.top_k(topk_scores.reshape(b, -1), k)
    topk_clses = (topk_ind // k).astype(jnp.float32)
    topk_inds = jnp.take_along_axis(topk_inds.reshape(b, -1), topk_ind, axis=1)
    topk_ys = jnp.take_along_axis(topk_ys.reshape(b, -1), topk_ind, axis=1)
    topk_xs = jnp.take_along_axis(topk_xs.reshape(b, -1), topk_ind, axis=1)
    return topk_score, topk_inds, topk_clses, topk_ys, topk_xs


def pairwise_iou(boxes1, boxes2):
    area1 = (boxes1[:, 2] - boxes1[:, 0]) * (boxes1[:, 3] - boxes1[:, 1])
    area2 = (boxes2[:, 2] - boxes2[:, 0]) * (boxes2[:, 3] - boxes2[:, 1])
    lt = jnp.maximum(boxes1[:, None, :2], boxes2[None, :, :2])
    rb = jnp.minimum(boxes1[:, None, 2:4], boxes2[None, :, 2:4])
    whi = jnp.clip(rb - lt, 0.0, None)
    inter = whi[:, :, 0] * whi[:, :, 1]
    union = area1[:, None] + area2[None, :] - inter
    return jnp.where(inter > 0, inter / union, 0.0)


def kernel(hm, wh, reg, gt_hm, gt_wh, gt_reg, reg_mask, target_box, ind):
    hm_s = _sigmoid(hm)
    nmsed = _nms(hm_s)
    scores, inds, clses, ys, xs = _topk(nmsed, K)

    reg_g = _transpose_and_gather_feat(reg, inds).reshape(B, K, 2)
    xs_f = xs.reshape(B, K, 1) + reg_g[:, :, 0:1]
    ys_f = ys.reshape(B, K, 1) + reg_g[:, :, 1:2]
    wh_g = _transpose_and_gather_feat(wh, inds).reshape(B, K, 2)
    bboxes = jnp.concatenate(
        [xs_f - wh_g[:, :, 0:1] / 2, ys_f - wh_g[:, :, 1:2] / 2,
         xs_f + wh_g[:, :, 0:1] / 2, ys_f + wh_g[:, :, 1:2] / 2], axis=2)

    ious = []
    for i in range(B):
        mask = target_box[:, -1] == i
        iou_all = pairwise_iou(bboxes[i], target_box[:, :4])
        ious.append(jnp.where(mask[None, :], iou_all, 0.0).max(axis=-1))
    iou = jnp.stack(ious, 0).reshape(-1)

    pos_sum, neg_sum, num_pos = _focal_sums(hm_s, gt_hm)

    b_idx = jnp.repeat(jnp.arange(B), K)
    cls_idx = jnp.clip(clses.astype(jnp.int32).reshape(-1), 0, C - 1)
    y_idx = jnp.clip(ys.astype(jnp.int32).reshape(-1), 0, H - 1)
    x_idx = jnp.clip(xs.astype(jnp.int32).reshape(-1), 0, W - 1)
    flat_pos = (b_idx * C + cls_idx) * (H * W) + y_idx * W + x_idx
    p_det = hm_s.reshape(-1)[flat_pos]
    g_old = gt_hm.reshape(-1)[flat_pos]
    g_new = jnp.clip(g_old + iou * 0.1, 0.0, 1.0)

    def terms(p, g):
        pos = (g == 1.0).astype(jnp.float32)
        neg = 1.0 - pos
        nw = (1.0 - g) ** 4
        pl_ = jnp.log(p) * (1.0 - p) ** 2 * pos
        nl_ = jnp.log(1.0 - p) * p * p * nw * neg
        return pl_, nl_, pos

    pl_o, nl_o, po_o = terms(p_det, g_old)
    pl_n, nl_n, po_n = terms(p_det, g_new)
    pos_sum = pos_sum + jnp.sum(pl_n - pl_o)
    neg_sum = neg_sum + jnp.sum(nl_n - nl_o)
    num_pos = num_pos + jnp.sum(po_n - po_o)

    hm_loss = jnp.where(num_pos == 0, -neg_sum, -(pos_sum + neg_sum) / jnp.maximum(num_pos, 1.0))

    def reg_l1(output, mask, index, target):
        pred = _transpose_and_gather_feat(output, index)
        mask_e = jnp.broadcast_to(mask[:, :, None], pred.shape).astype(pred.dtype)
        loss = jnp.abs(pred * mask_e - target * mask_e).sum()
        return loss / (mask_e.sum() + 1e-4)

    wh_loss = reg_l1(wh, reg_mask, ind, gt_wh)
    off_loss = reg_l1(reg, reg_mask, ind, gt_reg)
    return 1.0 * hm_loss + 0.1 * wh_loss + 1.0 * off_loss


# speedup vs baseline: 12.7055x; 12.4826x over previous
import functools

import jax
import jax.numpy as jnp
from jax import lax
from jax.experimental import pallas as pl
from jax.experimental.pallas import tpu as pltpu

B, C, H, W = 16, 80, 128, 128
K = 100
MAX_OBJS = 128
M = 512

PLANES = B * C
NROW = C * H
BLK = 16
HW = H * W


def _a_kernel(hm_ref, gt_ref, cmax_ref, carg_ref, cg_ref, part_ref):
    i = pl.program_id(0)
    x = hm_ref[...]
    g = gt_ref[...]
    s = jnp.clip(jax.nn.sigmoid(x), 1e-4, 1.0 - 1e-4)

    zc = jnp.zeros((BLK, H, 1), jnp.float32)
    left = jnp.concatenate([s[:, :, 1:], zc], axis=2)
    right = jnp.concatenate([zc, s[:, :, :-1]], axis=2)
    hx = jnp.maximum(jnp.maximum(left, right), s)
    zr = jnp.zeros((BLK, 1, W), jnp.float32)
    up = jnp.concatenate([hx[:, 1:, :], zr], axis=1)
    dn = jnp.concatenate([zr, hx[:, :-1, :]], axis=1)
    hmax = jnp.maximum(jnp.maximum(up, dn), hx)
    nm = jnp.where(hmax == s, s, 0.0)

    rmax = jnp.max(nm, axis=2)
    lane = lax.broadcasted_iota(jnp.int32, (BLK, H, W), 2)
    rarg = jnp.min(jnp.where(nm == rmax[:, :, None], lane, W), axis=2)
    onehot = lane == rarg[:, :, None]
    gsel = jnp.sum(jnp.where(onehot, g, 0.0), axis=2)
    cmax_ref[...] = rmax
    carg_ref[...] = rarg
    cg_ref[...] = gsel

    pos = (g == 1.0).astype(jnp.float32)
    one_m_g = 1.0 - g
    nw = one_m_g * one_m_g
    nw = nw * nw
    pos_loss = jnp.log(s) * (1.0 - s) * (1.0 - s) * pos
    neg_loss = jnp.log(1.0 - s) * s * s * nw * (1.0 - pos)

    def r(v):
        t = jnp.sum(v, axis=0)
        return jnp.sum(t.reshape(16, 8, 128), axis=0)

    part = jnp.stack([r(pos_loss), r(neg_loss), r(pos)], axis=0)

    @pl.when(i == 0)
    def _():
        part_ref[...] = jnp.zeros_like(part_ref)

    part_ref[...] += part


def _run_a(hm, gt_hm):
    return pl.pallas_call(
        _a_kernel,
        grid=(PLANES // BLK,),
        in_specs=[
            pl.BlockSpec((BLK, H, W), lambda i: (i, 0, 0)),
            pl.BlockSpec((BLK, H, W), lambda i: (i, 0, 0)),
        ],
        out_specs=[
            pl.BlockSpec((BLK, H), lambda i: (i, 0)),
            pl.BlockSpec((BLK, H), lambda i: (i, 0)),
            pl.BlockSpec((BLK, H), lambda i: (i, 0)),
            pl.BlockSpec((3, 8, 128), lambda i: (0, 0, 0)),
        ],
        out_shape=[
            jax.ShapeDtypeStruct((PLANES, H), jnp.float32),
            jax.ShapeDtypeStruct((PLANES, H), jnp.int32),
            jax.ShapeDtypeStruct((PLANES, H), jnp.float32),
            jax.ShapeDtypeStruct((3, 8, 128), jnp.float32),
        ],
    )(hm.reshape(PLANES, H, W), gt_hm.reshape(PLANES, H, W))


def _s_kernel(cm_ref, t_ref):
    v = cm_ref[...]
    bits = lax.bitcast_convert_type(v, jnp.int32)

    def body(_, lohi):
        lo, hi = lohi
        mid = (lo + hi) // 2
        cnt = jnp.sum((bits >= mid).astype(jnp.int32))
        take = cnt >= K
        return jnp.where(take, mid, lo), jnp.where(take, hi, mid)

    lo, _ = lax.fori_loop(0, 31, body, (jnp.int32(0), jnp.int32(0x3F800001)))
    t_ref[...] = jnp.full((1, 8, 128), lo, jnp.int32)


def _run_s(cmax):
    return pl.pallas_call(
        _s_kernel,
        grid=(B,),
        in_specs=[pl.BlockSpec((1, C, H), lambda i: (i, 0, 0))],
        out_specs=pl.BlockSpec((1, 8, 128), lambda i: (i, 0, 0)),
        out_shape=jax.ShapeDtypeStruct((B, 8, 128), jnp.int32),
    )(cmax.reshape(B, C, H))


def _d_kernel(part_ref, bx1_ref, by1_ref, bx2_ref, by2_ref, valid_ref,
              val_ref, gold_ref, tb_ref,
              pw0_ref, pw1_ref, pr0_ref, pr1_ref,
              tw0_ref, tw1_ref, tr0_ref, tr1_ref, rm_ref, o_ref):
    part = part_ref[...]
    pos_sum = jnp.sum(part[0])
    neg_sum = jnp.sum(part[1])
    num_pos = jnp.sum(part[2])

    tb = tb_ref[...]
    tx1 = tb[0]
    ty1 = tb[1]
    tx2 = tb[2]
    ty2 = tb[3]
    tbi = tb[4]
    a2 = (tx2 - tx1) * (ty2 - ty1)

    bx1 = bx1_ref[...]
    by1 = by1_ref[...]
    bx2 = bx2_ref[...]
    by2 = by2_ref[...]
    valid = valid_ref[...]

    ious = []
    for i in range(B):
        m = (tbi == float(i)).astype(jnp.float32)
        a1 = (bx2[i] - bx1[i]) * (by2[i] - by1[i])
        ltx = jnp.maximum(bx1[i][:, None], tx1[None, :])
        lty = jnp.maximum(by1[i][:, None], ty1[None, :])
        rbx = jnp.minimum(bx2[i][:, None], tx2[None, :])
        rby = jnp.minimum(by2[i][:, None], ty2[None, :])
        iw = jnp.maximum(rbx - ltx, 0.0)
        ih = jnp.maximum(rby - lty, 0.0)
        inter = iw * ih
        union = a1[:, None] + a2[None, :] - inter
        iou_all = jnp.where(inter > 0, inter / union, 0.0)
        ious.append(jnp.max(iou_all * m[None, :], axis=1))
    iou = jnp.stack(ious, axis=0) * valid

    p = jnp.clip(val_ref[...], 1e-4, 1.0 - 1e-4)
    g_old = jnp.clip(gold_ref[...], 0.0, 1.0)
    g_new = jnp.clip(g_old + iou * 0.1, 0.0, 1.0)

    def terms(pp, gg):
        po = (gg == 1.0).astype(jnp.float32)
        omg = 1.0 - gg
        nw = omg * omg
        nw = nw * nw
        t_pos = jnp.log(pp) * (1.0 - pp) * (1.0 - pp) * po
        t_neg = jnp.log(1.0 - pp) * pp * pp * nw * (1.0 - po)
        return t_pos, t_neg, po

    pl_o, nl_o, po_o = terms(p, g_old)
    pl_n, nl_n, po_n = terms(p, g_new)
    pos_sum = pos_sum + jnp.sum((pl_n - pl_o) * valid)
    neg_sum = neg_sum + jnp.sum((nl_n - nl_o) * valid)
    num_pos = num_pos + jnp.sum((po_n - po_o) * valid)
    hm_loss = jnp.where(num_pos == 0.0, -neg_sum,
                        -(pos_sum + neg_sum) / jnp.maximum(num_pos, 1.0))

    rm = rm_ref[...]
    wh_sum = (jnp.sum(jnp.abs(pw0_ref[...] * rm - tw0_ref[...] * rm)) +
              jnp.sum(jnp.abs(pw1_ref[...] * rm - tw1_ref[...] * rm)))
    off_sum = (jnp.sum(jnp.abs(pr0_ref[...] * rm - tr0_ref[...] * rm)) +
               jnp.sum(jnp.abs(pr1_ref[...] * rm - tr1_ref[...] * rm)))
    msum = 2.0 * jnp.sum(rm)
    wh_loss = wh_sum / (msum + 1e-4)
    off_loss = off_sum / (msum + 1e-4)
    total = hm_loss + 0.1 * wh_loss + 1.0 * off_loss
    o_ref[...] = jnp.broadcast_to(total, (1, 1))


def _run_d(part, bx1, by1, bx2, by2, valid, val, gold, tb8,
           pw0, pw1, pr0, pr1, tw0, tw1, tr0, tr1, rm):
    return pl.pallas_call(
        _d_kernel,
        out_shape=jax.ShapeDtypeStruct((1, 1), jnp.float32),
    )(part, bx1, by1, bx2, by2, valid, val, gold, tb8,
      pw0, pw1, pr0, pr1, tw0, tw1, tr0, tr1, rm)


def kernel(hm, wh, reg, gt_hm, gt_wh, gt_reg, reg_mask, target_box, ind):
    cmax, carg, cg, part = _run_a(hm, gt_hm)
    thr = _run_s(cmax)

    cmax_i = cmax.reshape(B, NROW)
    carg_i = carg.reshape(B, NROW)
    cg_i = cg.reshape(B, NROW)
    bits = lax.bitcast_convert_type(cmax_i, jnp.int32)
    sel = bits >= thr[:, 0, 0][:, None]
    rank = jnp.cumsum(sel.astype(jnp.int32), axis=1) - 1
    slot = jnp.where(sel, rank, MAX_OBJS)
    rowids = jnp.broadcast_to(jnp.arange(NROW, dtype=jnp.int32)[None, :],
                              (B, NROW))
    rows = jnp.zeros((B, MAX_OBJS), jnp.int32).at[
        jnp.arange(B)[:, None], slot].set(rowids, mode="drop")
    nsel = jnp.minimum(jnp.sum(sel, axis=1), MAX_OBJS)
    valid = (jnp.arange(MAX_OBJS)[None, :] < nsel[:, None]).astype(jnp.float32)

    take = jnp.take_along_axis
    val = take(cmax_i, rows, axis=1)
    argx = take(carg_i, rows, axis=1)
    gold = take(cg_i, rows, axis=1)
    y = rows % H
    x = argx
    flat = y * W + x
    wh_f = wh.reshape(B, 2, HW)
    reg_f = reg.reshape(B, 2, HW)
    w0 = take(wh_f[:, 0], flat, axis=1)
    w1 = take(wh_f[:, 1], flat, axis=1)
    r0 = take(reg_f[:, 0], flat, axis=1)
    r1 = take(reg_f[:, 1], flat, axis=1)
    xs = x.astype(jnp.float32) + r0
    ys = y.astype(jnp.float32) + r1
    bx1 = xs - w0 * 0.5
    by1 = ys - w1 * 0.5
    bx2 = xs + w0 * 0.5
    by2 = ys + w1 * 0.5

    tb8 = jnp.concatenate(
        [jnp.transpose(target_box), jnp.zeros((3, M), jnp.float32)], axis=0)

    indc = ind.astype(jnp.int32)
    pw0 = take(wh_f[:, 0], indc, axis=1)
    pw1 = take(wh_f[:, 1], indc, axis=1)
    pr0 = take(reg_f[:, 0], indc, axis=1)
    pr1 = take(reg_f[:, 1], indc, axis=1)
    tw0 = gt_wh[:, :, 0]
    tw1 = gt_wh[:, :, 1]
    tr0 = gt_reg[:, :, 0]
    tr1 = gt_reg[:, :, 1]

    out = _run_d(part, bx1, by1, bx2, by2, valid, val, gold, tb8,
                 pw0, pw1, pr0, pr1, tw0, tw1, tr0, tr1, reg_mask)
    return out[0, 0]
